# baseline (speedup 1.0000x reference)
import functools
import math

import jax
import jax.numpy as jnp
from jax import lax
from jax.experimental import pallas as pl
from jax.experimental.pallas import tpu as pltpu
from jax.experimental.pallas import tpu_sc as plsc

D_MODEL = 64
SCALE = math.sqrt(D_MODEL)
NBUF = 2
BLK = 512


@functools.lru_cache(maxsize=None)
def _build_convert(v: int, d: int):
    info = plsc.get_sparse_core_info()
    nc, ns = info.num_cores, info.num_subcores
    nw = nc * ns
    d2 = 2 * d
    nblk = v // BLK
    v_main = nblk * BLK

    mesh = plsc.VectorSubcoreMesh(core_axis_name="c", subcore_axis_name="s")

    @functools.partial(
        pl.kernel,
        mesh=mesh,
        out_type=jax.ShapeDtypeStruct((v // 2, d2), jnp.float32),
        scratch_types=[
            pltpu.VMEM((NBUF, d, BLK), jnp.float32),
            pltpu.VMEM((BLK // 2, d2), jnp.float32),
            pltpu.VMEM(((v - v_main) // 2, d2), jnp.float32),
            pltpu.SemaphoreType.DMA,
            pltpu.SemaphoreType.DMA,
        ],
        compiler_params=pltpu.CompilerParams(use_tc_tiling_on_sc=True,
                                             needs_layout_passes=False),
    )
    def convert(tt_hbm, tail_hbm, out_hbm, vbuf, obuf, tbuf, sem_i, sem_o):
        wid = lax.axis_index("s") * nc + lax.axis_index("c")
        my_nblk = (nblk - wid + nw - 1) // nw

        @pl.when(wid == 0)
        def _():
            pltpu.sync_copy(tail_hbm, tbuf)
            pltpu.sync_copy(tbuf, out_hbm.at[pl.ds(v_main // 2,
                                                   (v - v_main) // 2)])

        def blk_r0(i):
            return pl.multiple_of((wid + i * nw) * BLK, BLK)

        def fetch(i, slot):
            return pltpu.make_async_copy(
                tt_hbm.at[:, pl.ds(blk_r0(i), BLK)], vbuf.at[slot], sem_i)

        def put(i):
            return pltpu.make_async_copy(
                obuf,
                out_hbm.at[pl.ds(pl.multiple_of(blk_r0(i) // 2, BLK // 2),
                                 BLK // 2)],
                sem_o)

        @pl.when(my_nblk > 0)
        def _():
            fetch(0, 0).start()

        iota = jax.lax.broadcasted_iota(jnp.int32, (16,), 0)

        def blk_body(i, carry):
            slot = lax.rem(i, NBUF)

            @pl.when(i + 1 < my_nblk)
            def _():
                fetch(i + 1, 1 - slot).start()

            fetch(i, slot).wait()

            @pl.when(i >= 1)
            def _():
                put(i - 1).wait()

            def row_body(j4, c2):
                for dj in range(4):
                    j = j4 * 4 + dj
                    for k in range(d2 // 16):
                        col = 2 * j + (1 if k >= d // 16 else 0)
                        didx = iota + (k % (d // 16)) * 16
                        vals = plsc.load_gather(
                            vbuf.at[slot],
                            [didx, jnp.full((16,), col, jnp.int32)])
                        obuf[j, pl.ds(k * 16, 16)] = vals * SCALE
                return c2

            lax.fori_loop(0, BLK // 8, row_body, 0)
            put(i).start()
            return carry

        lax.fori_loop(0, my_nblk, blk_body, 0)

        @pl.when(my_nblk > 0)
        def _():
            put(my_nblk - 1).wait()

    return convert


@functools.lru_cache(maxsize=None)
def _build_gather(b: int, s: int, d: int):
    info = plsc.get_sparse_core_info()
    nc, ns = info.num_cores, info.num_subcores
    nw = nc * ns
    assert b % nw == 0 and s % 8 == 0
    b_per_w = b // nw
    n_per_w = b_per_w * s
    d2 = 2 * d

    mesh = plsc.VectorSubcoreMesh(core_axis_name="c", subcore_axis_name="s")

    @functools.partial(
        pl.kernel,
        mesh=mesh,
        out_type=jax.ShapeDtypeStruct((b, s, d), jnp.float32),
        scratch_types=[
            pltpu.VMEM((n_per_w,), jnp.int32),
            pltpu.VMEM((n_per_w,), jnp.int32),
            pltpu.VMEM((NBUF, s, d2), jnp.float32),
            pltpu.VMEM((NBUF, s, d), jnp.float32),
            pltpu.SemaphoreType.DMA,
            pltpu.SemaphoreType.DMA,
        ],
        compiler_params=pltpu.CompilerParams(use_tc_tiling_on_sc=True),
    )
    def lookup(idx_hbm, pairs_hbm, out_hbm, idx_v, jv, buf, obuf, sem_g,
               sem_o):
        wid = lax.axis_index("s") * nc + lax.axis_index("c")
        b0 = wid * b_per_w
        pltpu.sync_copy(idx_hbm.at[pl.ds(wid * n_per_w, n_per_w)], idx_v)

        def pair_body(i, carry):
            sl = pl.ds(i * 16, 16)
            jv[sl] = lax.shift_right_logical(idx_v[sl], 1)
            return carry

        lax.fori_loop(0, n_per_w // 16, pair_body, 0)

        def gather(c, slot):
            return pltpu.make_async_copy(
                pairs_hbm.at[jv.at[pl.ds(c * s, s)]], buf.at[slot], sem_g)

        def put(c, slot):
            return pltpu.make_async_copy(
                obuf.at[slot], out_hbm.at[b0 + c], sem_o)

        for c in range(NBUF):
            gather(c, c).start()

        def chunk_body(c, carry):
            slot = lax.rem(c, NBUF)
            gather(c, slot).wait()

            @pl.when(c >= NBUF)
            def _():
                put(c - NBUF, slot).wait()

            def row_body(j16, c2):
                start = lax.min(j16 * 16, s - 16)
                bases = (idx_v[pl.ds(c * s + start, 16)] & 1) * d
                for dj in range(16):
                    j = start + dj
                    base = bases[dj]
                    for k in range(d // 16):
                        obuf[slot, j, pl.ds(k * 16, 16)] = (
                            buf[slot, j, pl.ds(base + k * 16, 16)])
                return c2

            lax.fori_loop(0, (s + 15) // 16, row_body, 0)

            @pl.when(c + NBUF < b_per_w)
            def _():
                gather(c + NBUF, slot).start()

            put(c, slot).start()
            return carry

        lax.fori_loop(0, b_per_w, chunk_body, 0)

        for c in range(b_per_w - NBUF, b_per_w):
            put(c, c % NBUF).wait()

    return lookup


def kernel(x, table):
    b, s = x.shape
    v, d = table.shape
    v_main = (v // BLK) * BLK
    idx = x.reshape(b * s).astype(jnp.int32)
    tail = (table[v_main:] * SCALE).reshape((v - v_main) // 2, 2 * d)
    pairs = _build_convert(v, d)(table.T, tail)
    return _build_gather(b, s, d)(idx, pairs)


# speedup vs baseline: 1.1610x; 1.1610x over previous
import functools
import math

import jax
import jax.numpy as jnp
from jax import lax
from jax.experimental import pallas as pl
from jax.experimental.pallas import tpu as pltpu
from jax.experimental.pallas import tpu_sc as plsc

D_MODEL = 64
SCALE = math.sqrt(D_MODEL)
NBUF = 2
BLK = 512


@functools.lru_cache(maxsize=None)
def _build_convert(v: int, d: int):
    info = plsc.get_sparse_core_info()
    nc, ns = info.num_cores, info.num_subcores
    nw = nc * ns
    d2 = 2 * d
    nblk = v // BLK
    v_main = nblk * BLK

    mesh = plsc.VectorSubcoreMesh(core_axis_name="c", subcore_axis_name="s")

    @functools.partial(
        pl.kernel,
        mesh=mesh,
        out_type=jax.ShapeDtypeStruct((v // 2, d2), jnp.float32),
        scratch_types=[
            pltpu.VMEM((NBUF, d, BLK), jnp.float32),
            pltpu.VMEM((BLK // 2, d2), jnp.float32),
            pltpu.VMEM(((v - v_main) // 2, d2), jnp.float32),
            pltpu.SemaphoreType.DMA,
            pltpu.SemaphoreType.DMA,
        ],
        compiler_params=pltpu.CompilerParams(use_tc_tiling_on_sc=True,
                                             needs_layout_passes=False),
    )
    def convert(tt_hbm, tail_hbm, out_hbm, vbuf, obuf, tbuf, sem_i, sem_o):
        wid = lax.axis_index("s") * nc + lax.axis_index("c")
        my_nblk = (nblk - wid + nw - 1) // nw

        @pl.when(wid == 0)
        def _():
            pltpu.sync_copy(tail_hbm, tbuf)
            pltpu.sync_copy(tbuf, out_hbm.at[pl.ds(v_main // 2,
                                                   (v - v_main) // 2)])

        def blk_r0(i):
            return pl.multiple_of((wid + i * nw) * BLK, BLK)

        def fetch(i, slot):
            return pltpu.make_async_copy(
                tt_hbm.at[:, pl.ds(blk_r0(i), BLK)], vbuf.at[slot], sem_i)

        def put(i):
            return pltpu.make_async_copy(
                obuf,
                out_hbm.at[pl.ds(pl.multiple_of(blk_r0(i) // 2, BLK // 2),
                                 BLK // 2)],
                sem_o)

        @pl.when(my_nblk > 0)
        def _():
            fetch(0, 0).start()

        iota = jax.lax.broadcasted_iota(jnp.int32, (16,), 0)
        half = lax.shift_right_logical(iota, 1)
        par_d = (iota & 1) * d

        def blk_body(i, carry):
            slot = lax.rem(i, NBUF)

            @pl.when(i + 1 < my_nblk)
            def _():
                fetch(i + 1, 1 - slot).start()

            fetch(i, slot).wait()

            @pl.when(i >= 1)
            def _():
                put(i - 1).wait()

            def dd_body(dd, c2):
                col_i = par_d + dd

                def m_body(m16, c3):
                    for u in range(8):
                        m0 = (m16 * 8 + u) * 16
                        vals = vbuf[slot, dd, pl.ds(m0, 16)] * SCALE
                        plsc.store_scatter(obuf, [half + (m0 // 2), col_i],
                                           vals)
                    return c3

                lax.fori_loop(0, BLK // 128, m_body, 0)
                return c2

            lax.fori_loop(0, d, dd_body, 0)
            put(i).start()
            return carry

        lax.fori_loop(0, my_nblk, blk_body, 0)

        @pl.when(my_nblk > 0)
        def _():
            put(my_nblk - 1).wait()

    return convert


@functools.lru_cache(maxsize=None)
def _build_gather(b: int, s: int, d: int):
    info = plsc.get_sparse_core_info()
    nc, ns = info.num_cores, info.num_subcores
    nw = nc * ns
    assert b % nw == 0 and s % 8 == 0
    b_per_w = b // nw
    n_per_w = b_per_w * s
    d2 = 2 * d

    mesh = plsc.VectorSubcoreMesh(core_axis_name="c", subcore_axis_name="s")

    @functools.partial(
        pl.kernel,
        mesh=mesh,
        out_type=jax.ShapeDtypeStruct((b, s, d), jnp.float32),
        scratch_types=[
            pltpu.VMEM((n_per_w,), jnp.int32),
            pltpu.VMEM((n_per_w,), jnp.int32),
            pltpu.VMEM((NBUF, s, d2), jnp.float32),
            pltpu.VMEM((NBUF, s, d), jnp.float32),
            pltpu.SemaphoreType.DMA,
            pltpu.SemaphoreType.DMA,
        ],
        compiler_params=pltpu.CompilerParams(use_tc_tiling_on_sc=True),
    )
    def lookup(idx_hbm, pairs_hbm, out_hbm, idx_v, jv, buf, obuf, sem_g,
               sem_o):
        wid = lax.axis_index("s") * nc + lax.axis_index("c")
        b0 = wid * b_per_w
        pltpu.sync_copy(idx_hbm.at[pl.ds(wid * n_per_w, n_per_w)], idx_v)

        def pair_body(i, carry):
            sl = pl.ds(i * 16, 16)
            jv[sl] = lax.shift_right_logical(idx_v[sl], 1)
            return carry

        lax.fori_loop(0, n_per_w // 16, pair_body, 0)

        def gather(c, slot):
            return pltpu.make_async_copy(
                pairs_hbm.at[jv.at[pl.ds(c * s, s)]], buf.at[slot], sem_g)

        def put(c, slot):
            return pltpu.make_async_copy(
                obuf.at[slot], out_hbm.at[b0 + c], sem_o)

        for c in range(NBUF):
            gather(c, c).start()

        def chunk_body(c, carry):
            slot = lax.rem(c, NBUF)
            gather(c, slot).wait()

            @pl.when(c >= NBUF)
            def _():
                put(c - NBUF, slot).wait()

            def row_body(j16, c2):
                start = lax.min(j16 * 16, s - 16)
                bases = (idx_v[pl.ds(c * s + start, 16)] & 1) * d
                for dj in range(16):
                    j = start + dj
                    base = bases[dj]
                    for k in range(d // 16):
                        obuf[slot, j, pl.ds(k * 16, 16)] = (
                            buf[slot, j, pl.ds(base + k * 16, 16)])
                return c2

            lax.fori_loop(0, (s + 15) // 16, row_body, 0)

            @pl.when(c + NBUF < b_per_w)
            def _():
                gather(c + NBUF, slot).start()

            put(c, slot).start()
            return carry

        lax.fori_loop(0, b_per_w, chunk_body, 0)

        for c in range(b_per_w - NBUF, b_per_w):
            put(c, c % NBUF).wait()

    return lookup


def kernel(x, table):
    b, s = x.shape
    v, d = table.shape
    v_main = (v // BLK) * BLK
    idx = x.reshape(b * s).astype(jnp.int32)
    tail = (table[v_main:] * SCALE).reshape((v - v_main) // 2, 2 * d)
    pairs = _build_convert(v, d)(table.T, tail)
    return _build_gather(b, s, d)(idx, pairs)


# speedup vs baseline: 2.3633x; 2.0355x over previous
import functools
import math

import jax
import jax.numpy as jnp
from jax import lax
from jax.experimental import pallas as pl
from jax.experimental.pallas import tpu as pltpu
from jax.experimental.pallas import tpu_sc as plsc

D_MODEL = 64
SCALE = math.sqrt(D_MODEL)
NBUF = 2


@functools.lru_cache(maxsize=None)
def _build_lookup(b: int, s: int, d: int):
    info = plsc.get_sparse_core_info()
    nc, ns = info.num_cores, info.num_subcores
    nw = nc * ns
    assert b % nw == 0 and s % 8 == 0
    b_per_w = b // nw
    n_per_w = b_per_w * s
    d2 = 2 * d

    mesh = plsc.VectorSubcoreMesh(core_axis_name="c", subcore_axis_name="s")

    @functools.partial(
        pl.kernel,
        mesh=mesh,
        out_type=jax.ShapeDtypeStruct((b, s, d), jnp.float32),
        scratch_types=[
            pltpu.VMEM((n_per_w,), jnp.int32),
            pltpu.VMEM((n_per_w,), jnp.int32),
            pltpu.VMEM((NBUF, s, d2), jnp.float32),
            pltpu.VMEM((NBUF, s, d), jnp.float32),
            pltpu.SemaphoreType.DMA,
            pltpu.SemaphoreType.DMA,
        ],
        compiler_params=pltpu.CompilerParams(use_tc_tiling_on_sc=True),
    )
    def lookup(idx_hbm, pairs_hbm, out_hbm, idx_v, jv, buf, obuf, sem_g,
               sem_o):
        wid = lax.axis_index("s") * nc + lax.axis_index("c")
        b0 = wid * b_per_w
        pltpu.sync_copy(idx_hbm.at[pl.ds(wid * n_per_w, n_per_w)], idx_v)

        def pair_body(i, carry):
            sl = pl.ds(i * 16, 16)
            jv[sl] = lax.shift_right_logical(idx_v[sl], 1)
            return carry

        lax.fori_loop(0, n_per_w // 16, pair_body, 0)

        def gather(c, slot):
            return pltpu.make_async_copy(
                pairs_hbm.at[jv.at[pl.ds(c * s, s)]], buf.at[slot], sem_g)

        def put(c, slot):
            return pltpu.make_async_copy(
                obuf.at[slot], out_hbm.at[b0 + c], sem_o)

        for c in range(NBUF):
            gather(c, c).start()

        def chunk_body(c, carry):
            slot = lax.rem(c, NBUF)
            gather(c, slot).wait()

            @pl.when(c >= NBUF)
            def _():
                put(c - NBUF, slot).wait()

            def row_body(j16, c2):
                start = lax.min(j16 * 16, s - 16)
                bases = (idx_v[pl.ds(c * s + start, 16)] & 1) * d
                for dj in range(16):
                    j = start + dj
                    base = bases[dj]
                    for k in range(d // 16):
                        obuf[slot, j, pl.ds(k * 16, 16)] = (
                            buf[slot, j, pl.ds(base + k * 16, 16)] * SCALE)
                return c2

            lax.fori_loop(0, (s + 15) // 16, row_body, 0)

            @pl.when(c + NBUF < b_per_w)
            def _():
                gather(c + NBUF, slot).start()

            put(c, slot).start()
            return carry

        lax.fori_loop(0, b_per_w, chunk_body, 0)

        for c in range(b_per_w - NBUF, b_per_w):
            put(c, c % NBUF).wait()

    return lookup


def kernel(x, table):
    b, s = x.shape
    v, d = table.shape
    idx = x.reshape(b * s).astype(jnp.int32)
    pairs = table.reshape(v // 2, 2 * d)
    return _build_lookup(b, s, d)(idx, pairs)
